# Initial kernel scaffold; baseline (speedup 1.0000x reference)
import functools

import jax
import jax.numpy as jnp
from jax import lax
from jax.experimental import pallas as pl
from jax.experimental.pallas import tpu as pltpu
from jax.experimental.pallas import tpu_sc as plsc

N_NODES = 10000
D_FEAT = 256
D_OUT = 256

H = 128
NC = 2
NT = 16
CHUNK = 128
N_PAD = 10016
ROWS_PER_TILE = N_PAD // NT


def _sc_aggregate(x0, x1, colp, rowp, zrows, zdeg, onesrow, e_pad):
    edges_per_tile = e_pad // NT
    n_chunks = edges_per_tile // CHUNK
    mesh = plsc.VectorSubcoreMesh(core_axis_name="c", subcore_axis_name="s")

    @functools.partial(
        pl.kernel,
        out_type=[
            jax.ShapeDtypeStruct((N_PAD, H), jnp.float32),
            jax.ShapeDtypeStruct((N_PAD, H), jnp.float32),
            jax.ShapeDtypeStruct((N_PAD, 16), jnp.float32),
        ],
        mesh=mesh,
        scratch_types=[
            pltpu.VMEM_SHARED((N_PAD, H), jnp.float32),
            pltpu.VMEM_SHARED((N_PAD, 16), jnp.float32),
            pltpu.VMEM((CHUNK,), jnp.int32),
            pltpu.VMEM((CHUNK,), jnp.int32),
            pltpu.VMEM((CHUNK, H), jnp.float32),
            pltpu.VMEM((CHUNK, 16), jnp.float32),
            pltpu.SemaphoreType.DMA,
        ],
    )
    def sc_kernel(x0_h, x1_h, col_h, row_h, zrows_h, zdeg_h, ones_h,
                  agg0_h, agg1_h, deg_h,
                  acc_sp, deg_sp, colv, rowv, rows_v, ones_v, sem):
        cid = lax.axis_index("c")
        sid = lax.axis_index("s")
        rbase = sid * ROWS_PER_TILE

        pltpu.sync_copy(zrows_h, acc_sp.at[pl.ds(rbase, ROWS_PER_TILE)])

        @pl.when(cid == 0)
        def _():
            pltpu.sync_copy(zdeg_h, deg_sp.at[pl.ds(rbase, ROWS_PER_TILE)])

        pltpu.sync_copy(ones_h, ones_v)
        plsc.subcore_barrier()

        ebase = sid * edges_per_tile

        def chunk_body(k, _):
            base = ebase + k * CHUNK
            pltpu.sync_copy(col_h.at[pl.ds(base, CHUNK)], colv)
            pltpu.sync_copy(row_h.at[pl.ds(base, CHUNK)], rowv)

            @pl.when(cid == 0)
            def _():
                pltpu.async_copy(x0_h.at[colv], rows_v, sem).wait()

            @pl.when(cid == 1)
            def _():
                pltpu.async_copy(x1_h.at[colv], rows_v, sem).wait()

            pltpu.sync_copy(rows_v, acc_sp.at[rowv], add=True)

            @pl.when(cid == 0)
            def _():
                pltpu.sync_copy(ones_v, deg_sp.at[rowv], add=True)

            return 0

        lax.fori_loop(0, n_chunks, chunk_body, 0)
        plsc.subcore_barrier()

        sl = pl.ds(rbase, ROWS_PER_TILE)

        @pl.when(cid == 0)
        def _():
            pltpu.sync_copy(acc_sp.at[sl], agg0_h.at[sl])
            pltpu.sync_copy(deg_sp.at[sl], deg_h.at[sl])

        @pl.when(cid == 1)
        def _():
            pltpu.sync_copy(acc_sp.at[sl], agg1_h.at[sl])

    return sc_kernel(x0, x1, colp, rowp, zrows, zdeg, onesrow)


def _tc_linear(x, agg0, agg1, deg2d, wx, wa0, wa1, b2d):
    BM = 1000
    grid = N_NODES // BM

    def body(x_ref, a0_ref, a1_ref, d_ref, wx_ref, wa0_ref, wa1_ref, b_ref,
             o_ref):
        deg = jnp.maximum(d_ref[:, 0:1], 1.0)
        r = 1.0 / deg
        acc = jnp.dot(x_ref[...], wx_ref[...],
                      preferred_element_type=jnp.float32)
        acc = acc + jnp.dot(a0_ref[...] * r, wa0_ref[...],
                            preferred_element_type=jnp.float32)
        acc = acc + jnp.dot(a1_ref[...] * r, wa1_ref[...],
                            preferred_element_type=jnp.float32)
        o_ref[...] = acc + b_ref[...]

    return pl.pallas_call(
        body,
        grid=(grid,),
        in_specs=[
            pl.BlockSpec((BM, D_FEAT), lambda i: (i, 0)),
            pl.BlockSpec((BM, H), lambda i: (i, 0)),
            pl.BlockSpec((BM, H), lambda i: (i, 0)),
            pl.BlockSpec((BM, 16), lambda i: (i, 0)),
            pl.BlockSpec((D_FEAT, D_OUT), lambda i: (0, 0)),
            pl.BlockSpec((H, D_OUT), lambda i: (0, 0)),
            pl.BlockSpec((H, D_OUT), lambda i: (0, 0)),
            pl.BlockSpec((1, D_OUT), lambda i: (0, 0)),
        ],
        out_specs=pl.BlockSpec((BM, D_OUT), lambda i: (i, 0)),
        out_shape=jax.ShapeDtypeStruct((N_NODES, D_OUT), jnp.float32),
    )(x, agg0, agg1, deg2d, wx, wa0, wa1, b2d)


@jax.jit
def kernel(x, edge_index, W, b):
    n_edges = edge_index.shape[1]
    e_pad = ((n_edges + NT * CHUNK - 1) // (NT * CHUNK)) * (NT * CHUNK)

    row = edge_index[0]
    col = edge_index[1]
    pad_e = e_pad - n_edges
    rowp = jnp.concatenate([row, jnp.full((pad_e,), N_NODES, jnp.int32)])
    colp = jnp.concatenate([col, jnp.full((pad_e,), N_NODES, jnp.int32)])

    pad_n = N_PAD - N_NODES
    x0 = jnp.concatenate(
        [x[:, :H], jnp.zeros((pad_n, H), jnp.float32)], axis=0)
    x1 = jnp.concatenate(
        [x[:, H:], jnp.zeros((pad_n, H), jnp.float32)], axis=0)

    zrows = jnp.zeros((ROWS_PER_TILE, H), jnp.float32)
    zdeg = jnp.zeros((ROWS_PER_TILE, 16), jnp.float32)
    onesrow = jnp.zeros((CHUNK, 16), jnp.float32).at[:, 0].set(1.0)

    agg0, agg1, deg2d = _sc_aggregate(
        x0, x1, colp, rowp, zrows, zdeg, onesrow, e_pad)

    wx = W[:, :D_FEAT].T
    wa0 = W[:, D_FEAT:D_FEAT + H].T
    wa1 = W[:, D_FEAT + H:].T
    b2d = b[None, :]

    return _tc_linear(x, agg0, agg1, deg2d, wx, wa0, wa1, b2d)


# speedup vs baseline: 2.5275x; 2.5275x over previous
import functools

import jax
import jax.numpy as jnp
from jax import lax
from jax.experimental import pallas as pl
from jax.experimental.pallas import tpu as pltpu
from jax.experimental.pallas import tpu_sc as plsc

N_NODES = 10000
D_FEAT = 256
D_OUT = 256

H = 128
HA = 144
NC = 2
NT = 16
CHUNK = 128
N_PAD = 10112
ROWS_PER_TILE = N_PAD // NT


def _sc_aggregate(xh, colp, rowp, zrows, e_pad):
    edges_per_tile = e_pad // NT
    n_chunks = edges_per_tile // CHUNK
    mesh = plsc.VectorSubcoreMesh(core_axis_name="c", subcore_axis_name="s")

    @functools.partial(
        pl.kernel,
        out_type=jax.ShapeDtypeStruct((NC * N_PAD, HA), jnp.float32),
        mesh=mesh,
        scratch_types=[
            pltpu.VMEM((CHUNK,), jnp.int32),
            pltpu.VMEM((CHUNK,), jnp.int32),
            pltpu.VMEM((CHUNK, HA), jnp.float32),
            pltpu.VMEM_SHARED((N_PAD, HA), jnp.float32),
            pltpu.SemaphoreType.DMA,
        ],
        compiler_params=pltpu.CompilerParams(use_tc_tiling_on_sc=False),
    )
    def sc_kernel(xh_h, col_h, row_h, zrows_h, agg_h,
                  colv, rowv, rows_v, acc_sp, sem):
        cid = lax.axis_index("c")
        sid = lax.axis_index("s")
        rbase = sid * ROWS_PER_TILE
        out_base = cid * N_PAD + rbase
        sizes = [CHUNK] * 4 + [ROWS_PER_TILE - 4 * CHUNK]

        off = 0
        for sz in sizes:
            pltpu.sync_copy(zrows_h.at[pl.ds(off, sz)],
                            rows_v.at[pl.ds(0, sz)])
            pltpu.sync_copy(rows_v.at[pl.ds(0, sz)],
                            acc_sp.at[pl.ds(rbase + off, sz)])
            off += sz
        plsc.subcore_barrier()

        ebase = sid * edges_per_tile

        @pl.loop(0, n_chunks)
        def chunk_body(k):
            pltpu.sync_copy(col_h.at[pl.ds(cid * e_pad + ebase + k * CHUNK,
                                           CHUNK)], colv)
            pltpu.sync_copy(row_h.at[pl.ds(ebase + k * CHUNK, CHUNK)], rowv)
            pltpu.sync_copy(xh_h.at[colv], rows_v)
            pltpu.sync_copy(rows_v, acc_sp.at[rowv], add=True)

        plsc.subcore_barrier()

        off = 0
        for sz in sizes:
            pltpu.sync_copy(acc_sp.at[pl.ds(rbase + off, sz)],
                            rows_v.at[pl.ds(0, sz)])
            pltpu.sync_copy(rows_v.at[pl.ds(0, sz)],
                            agg_h.at[pl.ds(out_base + off, sz)])
            off += sz

    return sc_kernel(xh, colp, rowp, zrows)


def _tc_linear(x, agg, wx, wa0, wa1, b2d):
    BM = 1024
    grid = pl.cdiv(N_NODES, BM)

    def body(x_ref, a_ref, wx_ref, wa0_ref, wa1_ref, b_ref, o_ref):
        a0 = a_ref[0]
        a1 = a_ref[1]
        deg = jnp.maximum(a0[:, H:H + 1], 1.0)
        r = 1.0 / deg
        acc = jnp.dot(x_ref[...], wx_ref[...],
                      preferred_element_type=jnp.float32)
        acc = acc + jnp.dot(a0[:, :H] * r, wa0_ref[...],
                            preferred_element_type=jnp.float32)
        acc = acc + jnp.dot(a1[:, :H] * r, wa1_ref[...],
                            preferred_element_type=jnp.float32)
        o_ref[...] = acc + b_ref[...]

    return pl.pallas_call(
        body,
        grid=(grid,),
        in_specs=[
            pl.BlockSpec((BM, D_FEAT), lambda i: (i, 0)),
            pl.BlockSpec((NC, BM, HA), lambda i: (0, i, 0)),
            pl.BlockSpec((D_FEAT, D_OUT), lambda i: (0, 0)),
            pl.BlockSpec((H, D_OUT), lambda i: (0, 0)),
            pl.BlockSpec((H, D_OUT), lambda i: (0, 0)),
            pl.BlockSpec((1, D_OUT), lambda i: (0, 0)),
        ],
        out_specs=pl.BlockSpec((BM, D_OUT), lambda i: (i, 0)),
        out_shape=jax.ShapeDtypeStruct((N_NODES, D_OUT), jnp.float32),
    )(x, agg, wx, wa0, wa1, b2d)


@jax.jit
def kernel(x, edge_index, W, b):
    n_edges = edge_index.shape[1]
    e_pad = ((n_edges + NT * CHUNK - 1) // (NT * CHUNK)) * (NT * CHUNK)

    row = edge_index[0]
    col = edge_index[1]
    pad_e = e_pad - n_edges
    rowp = jnp.concatenate([row, jnp.full((pad_e,), N_NODES, jnp.int32)])
    colp = jnp.concatenate([col, jnp.full((pad_e,), N_NODES, jnp.int32)])
    colp = jnp.concatenate([colp, colp + N_PAD])

    xh = jnp.zeros((NC * N_PAD, HA), jnp.float32)
    xh = xh.at[:N_NODES, :H].set(x[:, :H])
    xh = xh.at[N_PAD:N_PAD + N_NODES, :H].set(x[:, H:])
    xh = xh.at[:N_NODES, H].set(1.0)
    xh = xh.at[N_PAD:N_PAD + N_NODES, H].set(1.0)

    zrows = jnp.zeros((ROWS_PER_TILE, HA), jnp.float32)

    agg2 = _sc_aggregate(xh, colp, rowp, zrows, e_pad)
    agg = agg2.reshape(NC, N_PAD, HA)

    wx = W[:, :D_FEAT].T
    wa0 = W[:, D_FEAT:D_FEAT + H].T
    wa1 = W[:, D_FEAT + H:].T
    b2d = b[None, :]

    return _tc_linear(x, agg, wx, wa0, wa1, b2d)
